# baseline (speedup 1.0000x reference)
import jax
import jax.numpy as jnp
from jax.experimental import pallas as pl

_NC = 21
_L = 5120


def _decode_tile(x_ref, o_ref):
    x = x_ref[...]
    o_ref[0:_NC] = x[0:_NC]
    dxy = x[21:23]
    dwh = x[23:25]
    axy = x[25:27]
    awh = x[27:29]
    vxy = x[29:31]
    vwh = x[31:33]
    c = dxy * awh * vxy + axy
    wh = jnp.exp(dwh * vwh) * awh
    cs = c * 512.0
    hs = wh * 256.0
    o_ref[21:23] = cs - hs
    o_ref[23:25] = cs + hs


def kernel(prediction):
    b, n, cin = prediction.shape
    xt = prediction.transpose(2, 0, 1)
    outt = pl.pallas_call(
        _decode_tile,
        grid=(pl.cdiv(n, _L),),
        in_specs=[pl.BlockSpec((cin, b, _L), lambda j: (0, 0, j))],
        out_specs=pl.BlockSpec((_NC + 4, b, _L), lambda j: (0, 0, j)),
        out_shape=jax.ShapeDtypeStruct((_NC + 4, b, n), jnp.float32),
    )(xt)
    return outt.transpose(1, 2, 0)
